# baseline (speedup 1.0000x reference)
import functools

import jax
import jax.numpy as jnp
from jax import lax
from jax.experimental import pallas as pl
from jax.experimental.pallas import tpu as pltpu
from jax.experimental.pallas import tpu_sc as plsc

IGNORE_LB = 255
NEG_LOG_THRESH = 0.35667494393873245

B, C, H, W = 8, 19, 512, 512
P = H * W
N = B * P
SH = 256
NB = H // SH

NW = 32
PER_W = N // NW


def _ce_body(lg_ref, lb_ref, loss_ref, nv_ref):
    first = jnp.logical_and(pl.program_id(0) == 0, pl.program_id(1) == 0)

    @pl.when(first)
    def _init():
        nv_ref[0, 0] = 0.0

    lab = lb_ref[0]
    x = [lg_ref[0, c] for c in range(C)]
    m = x[0]
    for c in range(1, C):
        m = jnp.maximum(m, x[c])
    s = jnp.exp(x[0] - m)
    sel = jnp.where(lab == 0, x[0], 0.0)
    for c in range(1, C):
        s = s + jnp.exp(x[c] - m)
        sel = sel + jnp.where(lab == c, x[c], 0.0)
    lse = jnp.log(s) + m
    valid = lab != IGNORE_LB
    loss_ref[0] = jnp.where(valid, lse - sel, 0.0)
    nv_ref[0, 0] += jnp.sum(valid.astype(jnp.float32))


def _ce_loss(logits, labels):
    loss, nv = pl.pallas_call(
        _ce_body,
        grid=(B, NB),
        in_specs=[
            pl.BlockSpec((1, C, SH, W), lambda b, j: (b, 0, j, 0)),
            pl.BlockSpec((1, SH, W), lambda b, j: (b, j, 0)),
        ],
        out_specs=[
            pl.BlockSpec((1, SH, W), lambda b, j: (b, j, 0)),
            pl.BlockSpec((1, 1), lambda b, j: (0, 0),
                         memory_space=pltpu.SMEM),
        ],
        out_shape=[
            jax.ShapeDtypeStruct((B, H, W), jnp.float32),
            jax.ShapeDtypeStruct((1, 1), jnp.float32),
        ],
    )(logits, labels.astype(jnp.int32))
    return loss, nv[0, 0]


def _sel_body(loss_hbm, t_hbm, out_hbm, buf, tbuf, vout):
    cid = lax.axis_index("c")
    sid = lax.axis_index("s")
    w = sid * 2 + cid
    b = w // 4
    h0 = (w % 4) * 128
    pltpu.sync_copy(t_hbm, tbuf)
    pltpu.sync_copy(loss_hbm.at[b, pl.ds(h0, H // 4), :], buf)
    tv = tbuf[...]

    def body(r, carry):
        cnt, sm = carry
        for c in range(W // 16):
            x = buf[r, pl.ds(c * 16, 16)]
            hard = x > tv
            cnt = cnt + jnp.where(hard, 1.0, 0.0)
            sm = sm + jnp.where(hard, x, 0.0)
        return cnt, sm

    zero = jnp.zeros((16,), jnp.float32)
    cnt, sm = lax.fori_loop(0, H // 4, body, (zero, zero))
    vout[pl.ds(0, 16)] = cnt
    vout[pl.ds(16, 16)] = sm
    pltpu.sync_copy(vout, out_hbm.at[w])


@functools.partial(
    pl.kernel,
    mesh=plsc.VectorSubcoreMesh(core_axis_name="c", subcore_axis_name="s"),
    out_type=jax.ShapeDtypeStruct((NW, 32), jnp.float32),
    scratch_types=[
        pltpu.VMEM((H // 4, W), jnp.float32),
        pltpu.VMEM((16,), jnp.float32),
        pltpu.VMEM((32,), jnp.float32),
    ],
)
def _sel_kernel(loss_hbm, t_hbm, out_hbm, buf, tbuf, vout):
    _sel_body(loss_hbm, t_hbm, out_hbm, buf, tbuf, vout)


def _count_sum(loss, t):
    part = _sel_kernel(loss, jnp.full((16,), t, jnp.float32))
    part = part.reshape(NW, 2, 16)
    return jnp.sum(part[:, 0, :]), jnp.sum(part[:, 1, :])


def _topk_sum(loss, k):
    def probe(v):
        t = lax.bitcast_convert_type(v, jnp.float32)
        c, s = _count_sum(loss, t)
        return t, c, s

    def cond(lh):
        return lh[0] < lh[1]

    def body(lh):
        lo, hi = lh
        mid = lo + (hi - lo) // 2
        _, c, _ = probe(mid)
        return lax.cond(c < k, lambda: (lo, mid), lambda: (mid + 1, hi))

    lo, hi = lax.while_loop(cond, body,
                            (jnp.int32(0), jnp.int32(0x7F800000)))
    t, c, s = probe(hi)
    return s + (k - c) * t


def kernel(logits, labels):
    loss, n_valid = _ce_loss(logits, labels)
    n_min = jnp.floor(n_valid / 16.0)
    n_hard, s_hard = _count_sum(loss, jnp.float32(NEG_LOG_THRESH))
    return lax.cond(n_hard >= n_min,
                    lambda: s_hard,
                    lambda: _topk_sum(loss, n_min))


# speedup vs baseline: 1.0253x; 1.0253x over previous
import functools

import jax
import jax.numpy as jnp
from jax import lax
from jax.experimental import pallas as pl
from jax.experimental.pallas import tpu as pltpu
from jax.experimental.pallas import tpu_sc as plsc

IGNORE_LB = 255
NEG_LOG_THRESH = 0.35667494393873245

B, C, H, W = 8, 19, 512, 512
P = H * W
N = B * P
SH = 128
NB = H // SH

NW = 32
PER_W = N // NW


def _ce_body(lg_ref, lb_ref, loss_ref, nv_ref):
    first = jnp.logical_and(pl.program_id(0) == 0, pl.program_id(1) == 0)

    @pl.when(first)
    def _init():
        nv_ref[0, 0] = 0.0

    lab = lb_ref[0]
    x = [lg_ref[0, c] for c in range(C)]
    m = x[0]
    for c in range(1, C):
        m = jnp.maximum(m, x[c])
    s = jnp.exp(x[0] - m)
    sel = jnp.where(lab == 0, x[0], 0.0)
    for c in range(1, C):
        s = s + jnp.exp(x[c] - m)
        sel = sel + jnp.where(lab == c, x[c], 0.0)
    lse = jnp.log(s) + m
    valid = lab != IGNORE_LB
    loss_ref[0] = jnp.where(valid, lse - sel, 0.0)
    nv_ref[0, 0] += jnp.sum(valid.astype(jnp.float32))


def _ce_loss(logits, labels):
    loss, nv = pl.pallas_call(
        _ce_body,
        grid=(B, NB),
        in_specs=[
            pl.BlockSpec((1, C, SH, W), lambda b, j: (b, 0, j, 0)),
            pl.BlockSpec((1, SH, W), lambda b, j: (b, j, 0)),
        ],
        out_specs=[
            pl.BlockSpec((1, SH, W), lambda b, j: (b, j, 0)),
            pl.BlockSpec((1, 1), lambda b, j: (0, 0),
                         memory_space=pltpu.SMEM),
        ],
        out_shape=[
            jax.ShapeDtypeStruct((B, H, W), jnp.float32),
            jax.ShapeDtypeStruct((1, 1), jnp.float32),
        ],
    )(logits, labels.astype(jnp.int32))
    return loss, nv[0, 0]


HR = H // 4
NCHUNK = 4
CR = HR // NCHUNK


def _sel_body(loss_hbm, t_hbm, out_hbm, buf, tbuf, vout, sems):
    cid = lax.axis_index("c")
    sid = lax.axis_index("s")
    w = sid * 2 + cid
    b = w // 4
    h0 = (w % 4) * HR
    copies = [
        pltpu.async_copy(loss_hbm.at[b, pl.ds(h0 + k * CR, CR), :],
                         buf.at[k % 2], sems.at[k % 2])
        for k in range(2)
    ]
    pltpu.sync_copy(t_hbm, tbuf)
    tv = tbuf[...]

    def chunk_loop(k, carry, bslot):
        def body(r, carry):
            cnt, sm = carry
            for c in range(W // 16):
                x = buf[bslot, r, pl.ds(c * 16, 16)]
                hard = x > tv
                cnt = cnt + jnp.where(hard, 1.0, 0.0)
                sm = sm + jnp.where(hard, x, 0.0)
            return cnt, sm

        return lax.fori_loop(0, CR, body, carry)

    zero = jnp.zeros((16,), jnp.float32)
    carry = (zero, zero)
    for k in range(NCHUNK):
        copies[k % 2].wait()
        if k + 2 < NCHUNK:
            copies[k % 2] = pltpu.async_copy(
                loss_hbm.at[b, pl.ds(h0 + (k + 2) * CR, CR), :],
                buf.at[k % 2], sems.at[k % 2])
        carry = chunk_loop(k, carry, k % 2)
    cnt, sm = carry
    vout[pl.ds(0, 16)] = cnt
    vout[pl.ds(16, 16)] = sm
    pltpu.sync_copy(vout, out_hbm.at[w])


@functools.partial(
    pl.kernel,
    mesh=plsc.VectorSubcoreMesh(core_axis_name="c", subcore_axis_name="s"),
    out_type=jax.ShapeDtypeStruct((NW, 32), jnp.float32),
    scratch_types=[
        pltpu.VMEM((2, CR, W), jnp.float32),
        pltpu.VMEM((16,), jnp.float32),
        pltpu.VMEM((32,), jnp.float32),
        pltpu.SemaphoreType.DMA((2,)),
    ],
)
def _sel_kernel(loss_hbm, t_hbm, out_hbm, buf, tbuf, vout, sems):
    _sel_body(loss_hbm, t_hbm, out_hbm, buf, tbuf, vout, sems)


def _count_sum(loss, t):
    part = _sel_kernel(loss, jnp.full((16,), t, jnp.float32))
    part = part.reshape(NW, 2, 16)
    return jnp.sum(part[:, 0, :]), jnp.sum(part[:, 1, :])


def _topk_sum(loss, k):
    def probe(v):
        t = lax.bitcast_convert_type(v, jnp.float32)
        c, s = _count_sum(loss, t)
        return t, c, s

    def cond(lh):
        return lh[0] < lh[1]

    def body(lh):
        lo, hi = lh
        mid = lo + (hi - lo) // 2
        _, c, _ = probe(mid)
        return lax.cond(c < k, lambda: (lo, mid), lambda: (mid + 1, hi))

    lo, hi = lax.while_loop(cond, body,
                            (jnp.int32(0), jnp.int32(0x7F800000)))
    t, c, s = probe(hi)
    return s + (k - c) * t


def kernel(logits, labels):
    loss, n_valid = _ce_loss(logits, labels)
    n_min = jnp.floor(n_valid / 16.0)
    n_hard, s_hard = _count_sum(loss, jnp.float32(NEG_LOG_THRESH))
    return lax.cond(n_hard >= n_min,
                    lambda: s_hard,
                    lambda: _topk_sum(loss, n_min))
